# baseline (speedup 1.0000x reference)
import functools

import jax
import jax.numpy as jnp
from jax import lax
from jax.experimental import pallas as pl
from jax.experimental.pallas import tpu as pltpu
from jax.experimental.pallas import tpu_sc as plsc


def _make_sc_embed(b_sz: int, t_len: int, d: int):
    info = plsc.get_sparse_core_info()
    nc, ns, nl = info.num_cores, info.num_subcores, info.num_lanes
    nw = nc * ns
    assert t_len % nw == 0
    tpw = t_len // nw
    assert b_sz % 2 == 0
    npair = b_sz // 2
    assert 2 * tpw <= 128
    assert tpw % 8 == 0
    assert d % nl == 0
    nvec = d // nl
    mesh = plsc.VectorSubcoreMesh(core_axis_name="c", subcore_axis_name="s")

    @functools.partial(
        pl.kernel,
        mesh=mesh,
        out_type=jax.ShapeDtypeStruct((b_sz, t_len, d), jnp.float32),
        scratch_types=[
            pltpu.VMEM((npair, 2 * tpw), jnp.int32),
            pltpu.VMEM((npair, 2 * tpw, d), jnp.float32),
            pltpu.VMEM((tpw, d), jnp.float32),
            pltpu.SemaphoreType.DMA,
            pltpu.SemaphoreType.DMA,
            pltpu.SemaphoreType.DMA,
            pltpu.SemaphoreType.DMA,
        ],
    )
    def k(x_hbm, table_hbm, pos_hbm, out_hbm, idx_v, rows_v, pos_v,
          isem, gsem, ssem, psem):
        wid = lax.axis_index("s") * nc + lax.axis_index("c")
        t0 = wid * tpw
        idx_copies = [
            pltpu.async_copy(x_hbm.at[2 * j + h, pl.ds(t0, tpw)],
                             idx_v.at[j, pl.ds(h * tpw, tpw)], isem)
            for j in range(npair) for h in range(2)
        ]
        gathers = []
        for j in range(npair):
            idx_copies[2 * j].wait()
            idx_copies[2 * j + 1].wait()
            gathers.append(
                pltpu.async_copy(table_hbm.at[idx_v.at[j]], rows_v.at[j],
                                 gsem))
        pos_copy = pltpu.async_copy(pos_hbm.at[pl.ds(t0, tpw)], pos_v, psem)
        pos_copy.wait()

        stores = []
        for j in range(npair):
            gathers[j].wait()
            for h in range(2):
                def add_row(r, _, j=j, h=h):
                    for c in range(nvec):
                        sl = pl.ds(c * nl, nl)
                        plsc.addupdate(rows_v.at[j, h * tpw + r, sl],
                                       pos_v[r, sl])
                    return 0

                lax.fori_loop(0, tpw, add_row, 0)
                stores.append(pltpu.async_copy(
                    rows_v.at[j, pl.ds(h * tpw, tpw)],
                    out_hbm.at[2 * j + h, pl.ds(t0, tpw)], ssem))
        for st in stores:
            st.wait()

    return k


def kernel(x, embedding, positional_encoding):
    b, t = x.shape
    v, d = embedding.shape
    fn = _make_sc_embed(b, t, d)
    return fn(x.astype(jnp.int32), embedding, positional_encoding)


# speedup vs baseline: 1.0017x; 1.0017x over previous
import functools

import jax
import jax.numpy as jnp
from jax import lax
from jax.experimental import pallas as pl
from jax.experimental.pallas import tpu as pltpu
from jax.experimental.pallas import tpu_sc as plsc


def _make_sc_embed(b_sz: int, t_len: int, d: int):
    info = plsc.get_sparse_core_info()
    nc, ns, nl = info.num_cores, info.num_subcores, info.num_lanes
    nw = nc * ns
    assert t_len % nw == 0
    tpw = t_len // nw
    assert b_sz % 2 == 0
    npair = b_sz // 2
    assert 2 * tpw <= 128
    assert tpw % 8 == 0
    assert d % nl == 0
    nvec = d // nl
    mesh = plsc.VectorSubcoreMesh(core_axis_name="c", subcore_axis_name="s")

    @functools.partial(
        pl.kernel,
        mesh=mesh,
        out_type=jax.ShapeDtypeStruct((b_sz, t_len, d), jnp.float32),
        scratch_types=[
            pltpu.VMEM((npair, 2 * tpw), jnp.int32),
            pltpu.VMEM((npair, 2 * tpw, d), jnp.float32),
            pltpu.VMEM((tpw, d), jnp.float32),
            pltpu.SemaphoreType.DMA,
            pltpu.SemaphoreType.DMA,
            pltpu.SemaphoreType.DMA,
            pltpu.SemaphoreType.DMA,
        ],
    )
    def k(x_hbm, table_hbm, pos_hbm, out_hbm, idx_v, rows_v, pos_v,
          isem, gsem, ssem, psem):
        wid = lax.axis_index("s") * nc + lax.axis_index("c")
        t0 = wid * tpw
        idx_copies = [
            pltpu.async_copy(x_hbm.at[2 * j + h, pl.ds(t0, tpw)],
                             idx_v.at[j, pl.ds(h * tpw, tpw)], isem)
            for j in range(npair) for h in range(2)
        ]
        gathers = []
        for j in range(npair):
            idx_copies[2 * j].wait()
            idx_copies[2 * j + 1].wait()
            gathers.append(
                pltpu.async_copy(table_hbm.at[idx_v.at[j]], rows_v.at[j],
                                 gsem))
        pos_copy = pltpu.async_copy(pos_hbm.at[pl.ds(t0, tpw)], pos_v, psem)
        pos_copy.wait()

        stores = []
        for j in range(npair):
            gathers[j].wait()

            def add_row(r, _, j=j):
                for c in range(nvec):
                    sl = pl.ds(c * nl, nl)
                    v = pos_v[r, sl]
                    plsc.addupdate(rows_v.at[j, r, sl], v)
                    plsc.addupdate(rows_v.at[j, tpw + r, sl], v)
                return 0

            lax.fori_loop(0, tpw, add_row, 0, unroll=2)
            for h in range(2):
                stores.append(pltpu.async_copy(
                    rows_v.at[j, pl.ds(h * tpw, tpw)],
                    out_hbm.at[2 * j + h, pl.ds(t0, tpw)], ssem))
        for st in stores:
            st.wait()

    return k


def kernel(x, embedding, positional_encoding):
    b, t = x.shape
    v, d = embedding.shape
    fn = _make_sc_embed(b, t, d)
    return fn(x.astype(jnp.int32), embedding, positional_encoding)


# speedup vs baseline: 1.0096x; 1.0079x over previous
import functools

import jax
import jax.numpy as jnp
from jax import lax
from jax.experimental import pallas as pl
from jax.experimental.pallas import tpu as pltpu
from jax.experimental.pallas import tpu_sc as plsc


def _make_sc_embed(b_sz: int, t_len: int, d: int):
    info = plsc.get_sparse_core_info()
    nc, ns, nl = info.num_cores, info.num_subcores, info.num_lanes
    nw = nc * ns
    assert t_len % nw == 0
    tpw = t_len // nw
    assert b_sz % 2 == 0
    npair = b_sz // 2
    assert 2 * tpw <= 128
    assert tpw % 16 == 0
    assert d % nl == 0
    nvec = d // nl
    chunks = [([2 * j, 2 * j + 1], tpw) for j in range(npair - 1)]
    chunks += [([b_sz - 2], tpw), ([b_sz - 1], tpw)]
    mesh = plsc.VectorSubcoreMesh(core_axis_name="c", subcore_axis_name="s")

    @functools.partial(
        pl.kernel,
        mesh=mesh,
        out_type=jax.ShapeDtypeStruct((b_sz, t_len, d), jnp.float32),
        scratch_types=[
            pltpu.VMEM((b_sz * tpw,), jnp.int32),
            pltpu.VMEM((b_sz * tpw, d), jnp.float32),
            pltpu.VMEM((tpw, d), jnp.float32),
            pltpu.SemaphoreType.DMA,
            pltpu.SemaphoreType.DMA,
            pltpu.SemaphoreType.DMA,
            pltpu.SemaphoreType.DMA,
        ],
    )
    def k(x_hbm, table_hbm, pos_hbm, out_hbm, idx_v, rows_v, pos_v,
          isem, gsem, ssem, psem):
        wid = lax.axis_index("s") * nc + lax.axis_index("c")
        t0 = wid * tpw
        bases = []
        b = 0
        for (brs, tl) in chunks:
            bases.append(b)
            b += len(brs) * tl
        idx_copies = []
        for (brs, tl), base in zip(chunks, bases):
            cps = [pltpu.async_copy(
                       x_hbm.at[br, pl.ds(t0, tl)],
                       idx_v.at[pl.ds(base + h * tl, tl)], isem)
                   for h, br in enumerate(brs)]
            idx_copies.append(cps)
        gathers = []
        for i, ((brs, tl), base) in enumerate(zip(chunks, bases)):
            for cp in idx_copies[i]:
                cp.wait()
            gathers.append(pltpu.async_copy(
                table_hbm.at[idx_v.at[pl.ds(base, len(brs) * tl)]],
                rows_v.at[pl.ds(base, len(brs) * tl)], gsem))
        pltpu.async_copy(pos_hbm.at[pl.ds(t0, tpw)], pos_v, psem).wait()

        stores = []
        for i, ((brs, tl), base) in enumerate(zip(chunks, bases)):
            gathers[i].wait()

            def add_row(r, _, base=base, tl=tl, nbr=len(brs)):
                for c in range(nvec):
                    sl = pl.ds(c * nl, nl)
                    v = pos_v[r, sl]
                    for h in range(nbr):
                        plsc.addupdate(rows_v.at[base + h * tl + r, sl], v)
                return 0

            lax.fori_loop(0, tl, add_row, 0)
            for h, br in enumerate(brs):
                stores.append(pltpu.async_copy(
                    rows_v.at[pl.ds(base + h * tl, tl)],
                    out_hbm.at[br, pl.ds(t0, tl)], ssem))
        for st in stores:
            st.wait()

    return k


def kernel(x, embedding, positional_encoding):
    b, t = x.shape
    v, d = embedding.shape
    fn = _make_sc_embed(b, t, d)
    return fn(x.astype(jnp.int32), embedding, positional_encoding)


# speedup vs baseline: 1.0214x; 1.0117x over previous
import functools

import jax
import jax.numpy as jnp
from jax import lax
from jax.experimental import pallas as pl
from jax.experimental.pallas import tpu as pltpu
from jax.experimental.pallas import tpu_sc as plsc


def _make_sc_embed(b_sz: int, t_len: int, d: int):
    info = plsc.get_sparse_core_info()
    nc, ns, nl = info.num_cores, info.num_subcores, info.num_lanes
    nw = nc * ns
    assert t_len % nw == 0
    tpw = t_len // nw
    assert b_sz % 2 == 0
    npair = b_sz // 2
    assert 2 * tpw <= 128
    assert tpw % 8 == 0
    assert d % nl == 0
    nvec = d // nl
    mesh = plsc.VectorSubcoreMesh(core_axis_name="c", subcore_axis_name="s")

    @functools.partial(
        pl.kernel,
        mesh=mesh,
        out_type=jax.ShapeDtypeStruct((b_sz, t_len, d), jnp.float32),
        scratch_types=[
            pltpu.VMEM((npair, 2 * tpw), jnp.int32),
            pltpu.VMEM((npair, 2 * tpw, d), jnp.float32),
            pltpu.VMEM((tpw, d), jnp.float32),
            pltpu.SemaphoreType.DMA,
            pltpu.SemaphoreType.DMA,
            pltpu.SemaphoreType.DMA,
            pltpu.SemaphoreType.DMA,
        ],
    )
    def k(x_hbm, table_hbm, pos_hbm, out_hbm, idx_v, rows_v, pos_v,
          isem, gsem, ssem, psem):
        wid = lax.axis_index("s") * nc + lax.axis_index("c")
        t0 = wid * tpw
        idx_copies = [
            pltpu.async_copy(x_hbm.at[2 * j + h, pl.ds(t0, tpw)],
                             idx_v.at[j, pl.ds(h * tpw, tpw)], isem)
            for j in range(npair) for h in range(2)
        ]
        gathers = []
        for j in range(npair):
            idx_copies[2 * j].wait()
            idx_copies[2 * j + 1].wait()
            gathers.append(
                pltpu.async_copy(table_hbm.at[idx_v.at[j]], rows_v.at[j],
                                 gsem))
        pos_copy = pltpu.async_copy(pos_hbm.at[pl.ds(t0, tpw)], pos_v, psem)
        pos_copy.wait()

        stores = []
        for j in range(npair):
            gathers[j].wait()

            def add_row(r, _, j=j):
                for c in range(nvec):
                    sl = pl.ds(c * nl, nl)
                    v = pos_v[r, sl]
                    plsc.addupdate(rows_v.at[j, r, sl], v)
                    plsc.addupdate(rows_v.at[j, tpw + r, sl], v)
                return 0

            lax.fori_loop(0, tpw, add_row, 0)
            for h in range(2):
                stores.append(pltpu.async_copy(
                    rows_v.at[j, pl.ds(h * tpw, tpw)],
                    out_hbm.at[2 * j + h, pl.ds(t0, tpw)], ssem))
        for st in stores:
            st.wait()

    return k


def kernel(x, embedding, positional_encoding):
    b, t = x.shape
    v, d = embedding.shape
    fn = _make_sc_embed(b, t, d)
    return fn(x.astype(jnp.int32), embedding, positional_encoding)
